# baseline (speedup 1.0000x reference)
import jax
import jax.numpy as jnp
from jax.experimental import pallas as pl
from jax.experimental.pallas import tpu as pltpu

TH = 64


def _gmul(a, m):
    R, M = a.shape
    return (a.reshape(R // 8, 8, M) * m[None]).reshape(R, M)


def _body(xm_ref, xu_ref, xd_ref, mm_ref, mu_ref, md_ref,
          w1_ref, b1_ref, w2_ref, b2_ref, w3_ref, b3_ref, o_ref):
    t = pl.program_id(1)
    nt = pl.num_programs(1)
    C, N = xm_ref.shape[1], xm_ref.shape[2]
    W = xu_ref.shape[2]
    Cm = w1_ref.shape[0]
    Ne = N + 2 * W

    xmf = xm_ref[0]
    m8 = mm_ref[0]

    t1m = jnp.dot(w1_ref[...], _gmul(xmf, m8), preferred_element_type=jnp.float32)
    t1m = _gmul(jnp.maximum(t1m + b1_ref[...], 0.0), m8)

    xu, xd = xu_ref[0], xd_ref[0]
    mu8, md8 = mu_ref[0], md_ref[0]
    t1u = jnp.dot(w1_ref[...], _gmul(xu, mu8), preferred_element_type=jnp.float32)
    t1u = _gmul(jnp.maximum(t1u + b1_ref[...], 0.0), mu8)
    t1u = t1u * jnp.where(t == 0, 0.0, 1.0)
    t1d = jnp.dot(w1_ref[...], _gmul(xd, md8), preferred_element_type=jnp.float32)
    t1d = _gmul(jnp.maximum(t1d + b1_ref[...], 0.0), md8)
    t1d = t1d * jnp.where(t == nt - 1, 0.0, 1.0)

    t1e = jnp.concatenate([t1u, t1m, t1d], axis=1)

    col = jax.lax.broadcasted_iota(jnp.int32, (8, Ne), 1) % W
    zR = jnp.where(col == 0, 0.0, 1.0)
    zL = jnp.where(col == W - 1, 0.0, 1.0)
    zero1 = jnp.zeros((Cm, 1), jnp.float32)
    t1eR = _gmul(jnp.concatenate([zero1, t1e[:, :Ne - 1]], axis=1), zR)
    t1eL = _gmul(jnp.concatenate([t1e[:, 1:], zero1], axis=1), zL)

    acc = b2_ref[...] * jnp.ones((Cm, N), jnp.float32)
    for dy in range(3):
        for dx, src in ((0, t1eR), (1, t1e), (2, t1eL)):
            sl = src[:, dy * W:dy * W + N]
            acc = acc + jnp.dot(w2_ref[dy * 3 + dx], sl,
                                preferred_element_type=jnp.float32)
    t2 = _gmul(jnp.maximum(acc, 0.0), m8)

    out = jnp.dot(w3_ref[...], t2, preferred_element_type=jnp.float32) + b3_ref[...]
    o_ref[0] = jnp.maximum(out + xmf, 0.0)


def kernel(x, mask, w1, g1, b1, rm1, rv1, w2, g2, b2, rm2, rv2,
           w3, g3, b3, rm3, rv3, inference=False):
    B, C, H, W = x.shape
    Cm = w1.shape[0]
    mh, mw = mask.shape[2], mask.shape[3]
    N = TH * W

    s1 = g1 / jnp.sqrt(rv1 + 1e-5)
    s2 = g2 / jnp.sqrt(rv2 + 1e-5)
    s3 = g3 / jnp.sqrt(rv3 + 1e-5)
    w1f = w1[:, :, 0, 0] * s1[:, None]
    b1f = (b1 - rm1 * s1)[:, None]
    w2f = jnp.transpose(w2 * s2[:, None, None, None], (2, 3, 0, 1)).reshape(9, Cm, Cm)
    b2f = (b2 - rm2 * s2)[:, None]
    w3f = w3[:, :, 0, 0] * s3[:, None]
    b3f = (b3 - rm3 * s3)[:, None]

    mfull = jnp.broadcast_to(mask[:, 0, :, None, :, None],
                             (B, mh, H // mh, mw, W // mw)).reshape(B, 1, H * W)
    m8 = jnp.broadcast_to(mfull, (B, 8, H * W))

    xr = x.reshape(B, C, H * W)
    nt = H // TH
    grid = (B, nt)

    out = pl.pallas_call(
        _body,
        grid=grid,
        in_specs=[
            pl.BlockSpec((1, C, N), lambda b, t: (b, 0, t)),
            pl.BlockSpec((1, C, W), lambda b, t: (b, 0, jnp.maximum(t * TH - 1, 0))),
            pl.BlockSpec((1, C, W), lambda b, t: (b, 0, jnp.minimum(t * TH + TH, H - 1))),
            pl.BlockSpec((1, 8, N), lambda b, t: (b, 0, t)),
            pl.BlockSpec((1, 8, W), lambda b, t: (b, 0, jnp.maximum(t * TH - 1, 0))),
            pl.BlockSpec((1, 8, W), lambda b, t: (b, 0, jnp.minimum(t * TH + TH, H - 1))),
            pl.BlockSpec((Cm, C), lambda b, t: (0, 0)),
            pl.BlockSpec((Cm, 1), lambda b, t: (0, 0)),
            pl.BlockSpec((9, Cm, Cm), lambda b, t: (0, 0, 0)),
            pl.BlockSpec((Cm, 1), lambda b, t: (0, 0)),
            pl.BlockSpec((C, Cm), lambda b, t: (0, 0)),
            pl.BlockSpec((C, 1), lambda b, t: (0, 0)),
        ],
        out_specs=pl.BlockSpec((1, C, N), lambda b, t: (b, 0, t)),
        out_shape=jax.ShapeDtypeStruct((B, C, H * W), jnp.float32),
        compiler_params=pltpu.CompilerParams(
            dimension_semantics=("parallel", "arbitrary")),
    )(xr, xr, xr, m8, m8, m8, w1f, b1f, w2f, b2f, w3f, b3f)
    return out.reshape(B, C, H, W)


# speedup vs baseline: 2.1146x; 2.1146x over previous
import jax
import jax.numpy as jnp
from jax.experimental import pallas as pl
from jax.experimental.pallas import tpu as pltpu

TH = 64
HB = 8


def _gmul(a, m):
    R, M = a.shape
    return (a.reshape(R // 8, 8, M) * m[None]).reshape(R, M)


def _body(xm_ref, xu_ref, xd_ref, mm_ref, mu_ref, md_ref,
          w1_ref, b1_ref, w2_ref, b2_ref, w3_ref, b3_ref, o_ref):
    t = pl.program_id(1)
    nt = pl.num_programs(1)
    C = xm_ref.shape[1]
    N = xm_ref.shape[2] * xm_ref.shape[3]
    W = xu_ref.shape[3]
    Cm = w1_ref.shape[0]
    Ne = N + 2 * W

    xmf = xm_ref[0].reshape(C, N)
    m8 = mm_ref[0]

    t1m = jnp.dot(w1_ref[...], _gmul(xmf, m8), preferred_element_type=jnp.float32)
    t1m = _gmul(jnp.maximum(t1m + b1_ref[...], 0.0), m8)

    xu = xu_ref[0, :, HB - 1]
    xd = xd_ref[0, :, 0]
    mu8, md8 = mu_ref[0], md_ref[0]
    t1u = jnp.dot(w1_ref[...], _gmul(xu, mu8), preferred_element_type=jnp.float32)
    t1u = _gmul(jnp.maximum(t1u + b1_ref[...], 0.0), mu8)
    t1u = t1u * jnp.where(t == 0, 0.0, 1.0)
    t1d = jnp.dot(w1_ref[...], _gmul(xd, md8), preferred_element_type=jnp.float32)
    t1d = _gmul(jnp.maximum(t1d + b1_ref[...], 0.0), md8)
    t1d = t1d * jnp.where(t == nt - 1, 0.0, 1.0)

    t1e = jnp.concatenate([t1u, t1m, t1d], axis=1)

    col = jax.lax.broadcasted_iota(jnp.int32, (8, Ne), 1) % W
    zR = jnp.where(col == 0, 0.0, 1.0)
    zL = jnp.where(col == W - 1, 0.0, 1.0)
    zero1 = jnp.zeros((Cm, 1), jnp.float32)
    t1eR = _gmul(jnp.concatenate([zero1, t1e[:, :Ne - 1]], axis=1), zR)
    t1eL = _gmul(jnp.concatenate([t1e[:, 1:], zero1], axis=1), zL)

    acc = b2_ref[...] * jnp.ones((Cm, N), jnp.float32)
    for dy in range(3):
        for dx, src in ((0, t1eR), (1, t1e), (2, t1eL)):
            sl = src[:, dy * W:dy * W + N]
            acc = acc + jnp.dot(w2_ref[dy * 3 + dx], sl,
                                preferred_element_type=jnp.float32)
    t2 = _gmul(jnp.maximum(acc, 0.0), m8)

    out = jnp.dot(w3_ref[...], t2, preferred_element_type=jnp.float32) + b3_ref[...]
    Th = N // W
    o_ref[0] = jnp.maximum(out + xmf, 0.0).reshape(C, Th, W)


def kernel(x, mask, w1, g1, b1, rm1, rv1, w2, g2, b2, rm2, rv2,
           w3, g3, b3, rm3, rv3, inference=False):
    B, C, H, W = x.shape
    Cm = w1.shape[0]
    mh, mw = mask.shape[2], mask.shape[3]
    N = TH * W

    s1 = g1 / jnp.sqrt(rv1 + 1e-5)
    s2 = g2 / jnp.sqrt(rv2 + 1e-5)
    s3 = g3 / jnp.sqrt(rv3 + 1e-5)
    w1f = w1[:, :, 0, 0] * s1[:, None]
    b1f = (b1 - rm1 * s1)[:, None]
    w2f = jnp.transpose(w2 * s2[:, None, None, None], (2, 3, 0, 1)).reshape(9, Cm, Cm)
    b2f = (b2 - rm2 * s2)[:, None]
    w3f = w3[:, :, 0, 0] * s3[:, None]
    b3f = (b3 - rm3 * s3)[:, None]

    mfull = jnp.broadcast_to(mask[:, 0, :, None, :, None],
                             (B, mh, H // mh, mw, W // mw)).reshape(B, 1, H * W)
    m8 = jnp.broadcast_to(mfull, (B, 8, H * W))

    nt = H // TH
    nhb = H // HB
    rb = TH // HB
    grid = (B, nt)

    out = pl.pallas_call(
        _body,
        grid=grid,
        in_specs=[
            pl.BlockSpec((1, C, TH, W), lambda b, t: (b, 0, t, 0)),
            pl.BlockSpec((1, C, HB, W), lambda b, t: (b, 0, jnp.maximum(t * rb - 1, 0), 0)),
            pl.BlockSpec((1, C, HB, W), lambda b, t: (b, 0, jnp.minimum(t * rb + rb, nhb - 1), 0)),
            pl.BlockSpec((1, 8, N), lambda b, t: (b, 0, t)),
            pl.BlockSpec((1, 8, W), lambda b, t: (b, 0, jnp.maximum(t * TH - 1, 0))),
            pl.BlockSpec((1, 8, W), lambda b, t: (b, 0, jnp.minimum(t * TH + TH, H - 1))),
            pl.BlockSpec((Cm, C), lambda b, t: (0, 0)),
            pl.BlockSpec((Cm, 1), lambda b, t: (0, 0)),
            pl.BlockSpec((9, Cm, Cm), lambda b, t: (0, 0, 0)),
            pl.BlockSpec((Cm, 1), lambda b, t: (0, 0)),
            pl.BlockSpec((C, Cm), lambda b, t: (0, 0)),
            pl.BlockSpec((C, 1), lambda b, t: (0, 0)),
        ],
        out_specs=pl.BlockSpec((1, C, TH, W), lambda b, t: (b, 0, t, 0)),
        out_shape=jax.ShapeDtypeStruct((B, C, H, W), jnp.float32),
        compiler_params=pltpu.CompilerParams(
            dimension_semantics=("parallel", "arbitrary")),
    )(x, x, x, m8, m8, m8, w1f, b1f, w2f, b2f, w3f, b3f)
    return out
